# baseline (speedup 1.0000x reference)
import functools

import jax
import jax.numpy as jnp
from jax.experimental import pallas as pl
from jax.experimental.pallas import tpu as pltpu

HIDDEN = 1024
N_EXPERTS = 8
INTER = 512
SHARED_DIM = 512


def _dotT(a, b):
    return jax.lax.dot_general(a, b, (((1,), (1,)), ((), ())),
                               preferred_element_type=jnp.float32)


def _moe_body(x_ref, gate_ref, w1_ref, w2_ref, wgu_ref, wd_ref,
              out_ref, xbf_ref, comb_ref, h_ref, s_ref):
    e = pl.program_id(0)

    @pl.when(e == 0)
    def _():
        x = x_ref[...]
        xbf_ref[...] = x.astype(jnp.bfloat16)
        logits = _dotT(x, gate_ref[...])
        m = jnp.max(logits, axis=-1, keepdims=True)
        ex = jnp.exp(logits - m)
        p = ex / jnp.sum(ex, axis=-1, keepdims=True)
        idx = jax.lax.broadcasted_iota(jnp.int32, p.shape, 1)
        m1 = jnp.max(p, axis=-1, keepdims=True)
        i1 = jnp.min(jnp.where(p == m1, idx, N_EXPERTS), axis=-1, keepdims=True)
        oh1 = (idx == i1)
        p2 = jnp.where(oh1, -jnp.inf, p)
        m2 = jnp.max(p2, axis=-1, keepdims=True)
        i2 = jnp.min(jnp.where(p2 == m2, idx, N_EXPERTS), axis=-1, keepdims=True)
        oh2 = (idx == i2)
        denom = m1 + m2
        comb_ref[...] = (jnp.where(oh1, m1 / denom, 0.0)
                         + jnp.where(oh2, m2 / denom, 0.0))

    @pl.when(jnp.logical_and(e >= 1, e <= N_EXPERTS))
    def _():
        y = _dotT(h_ref[...], w2_ref[0].astype(jnp.bfloat16))
        oh_e = (jax.lax.broadcasted_iota(jnp.int32, (1, N_EXPERTS), 1)
                == e - 1)
        w_col = jnp.sum(jnp.where(oh_e, comb_ref[...], 0.0),
                        axis=-1, keepdims=True)

        @pl.when(e == 1)
        def _():
            out_ref[...] = w_col * y

        @pl.when(e != 1)
        def _():
            out_ref[...] += w_col * y

    @pl.when(e < N_EXPERTS)
    def _():
        h = _dotT(xbf_ref[...], w1_ref[0].astype(jnp.bfloat16))
        h_ref[...] = (h * jax.nn.sigmoid(h)).astype(jnp.bfloat16)

    @pl.when(e == N_EXPERTS)
    def _():
        gu = _dotT(xbf_ref[...], wgu_ref[...].astype(jnp.bfloat16))
        gate = gu[:, :SHARED_DIM]
        up = gu[:, SHARED_DIM:]
        s_ref[...] = (gate * jax.nn.sigmoid(gate) * up).astype(jnp.bfloat16)

    @pl.when(e == N_EXPERTS + 1)
    def _():
        out_ref[...] += _dotT(s_ref[...], wd_ref[...].astype(jnp.bfloat16))


def kernel(hidden_states, gate_w, w1, w2, shared_gate_up_w, shared_down_w):
    orig_shape = hidden_states.shape
    T = orig_shape[0] * orig_shape[1]
    x2d = hidden_states.reshape(T, HIDDEN)

    out = pl.pallas_call(
        _moe_body,
        grid=(N_EXPERTS + 2,),
        in_specs=[
            pl.BlockSpec((T, HIDDEN), lambda e: (0, 0)),
            pl.BlockSpec((N_EXPERTS, HIDDEN), lambda e: (0, 0)),
            pl.BlockSpec((1, INTER, HIDDEN),
                         lambda e: (jnp.minimum(e, N_EXPERTS - 1), 0, 0)),
            pl.BlockSpec((1, HIDDEN, INTER),
                         lambda e: (jnp.clip(e - 1, 0, N_EXPERTS - 1), 0, 0)),
            pl.BlockSpec((2 * SHARED_DIM, HIDDEN), lambda e: (0, 0)),
            pl.BlockSpec((HIDDEN, SHARED_DIM), lambda e: (0, 0)),
        ],
        out_specs=pl.BlockSpec((T, HIDDEN), lambda e: (0, 0)),
        out_shape=jax.ShapeDtypeStruct((T, HIDDEN), jnp.float32),
        scratch_shapes=[
            pltpu.VMEM((T, HIDDEN), jnp.bfloat16),
            pltpu.VMEM((T, N_EXPERTS), jnp.float32),
            pltpu.VMEM((T, INTER), jnp.bfloat16),
            pltpu.VMEM((T, SHARED_DIM), jnp.bfloat16),
        ],
    )(x2d, gate_w, w1, w2, shared_gate_up_w, shared_down_w)

    return out.reshape(orig_shape)


# speedup vs baseline: 1.1165x; 1.1165x over previous
import functools

import jax
import jax.numpy as jnp
from jax.experimental import pallas as pl
from jax.experimental.pallas import tpu as pltpu

HIDDEN = 1024
N_EXPERTS = 8
INTER = 512
SHARED_DIM = 512


def _dotT(a, b):
    return jax.lax.dot_general(a, b, (((1,), (1,)), ((), ())),
                               preferred_element_type=jnp.float32)


def _moe_body(x_ref, gate_ref, w1_ref, w2_ref, wgu_ref, wd_ref,
              out_ref, xbf_ref, comb_ref, h_ref, s_ref):
    e = pl.program_id(0)

    @pl.when(e == 0)
    def _():
        x = x_ref[...]
        xbf = x.astype(jnp.bfloat16)
        xbf_ref[...] = xbf
        logits = _dotT(x, gate_ref[...])
        m = jnp.max(logits, axis=-1, keepdims=True)
        ex = jnp.exp(logits - m)
        p = ex / jnp.sum(ex, axis=-1, keepdims=True)
        idx = jax.lax.broadcasted_iota(jnp.int32, p.shape, 1)
        m1 = jnp.max(p, axis=-1, keepdims=True)
        i1 = jnp.min(jnp.where(p == m1, idx, N_EXPERTS), axis=-1, keepdims=True)
        oh1 = (idx == i1)
        p2 = jnp.where(oh1, -jnp.inf, p)
        m2 = jnp.max(p2, axis=-1, keepdims=True)
        i2 = jnp.min(jnp.where(p2 == m2, idx, N_EXPERTS), axis=-1, keepdims=True)
        oh2 = (idx == i2)
        denom = m1 + m2
        comb_ref[...] = (jnp.where(oh1, m1 / denom, 0.0)
                         + jnp.where(oh2, m2 / denom, 0.0))
        gu = _dotT(xbf, wgu_ref[...].astype(jnp.bfloat16))
        gate = gu[:, :SHARED_DIM]
        up = gu[:, SHARED_DIM:]
        s_ref[...] = (gate * jax.nn.sigmoid(gate) * up).astype(jnp.bfloat16)

    @pl.when(e == 1)
    def _():
        out_ref[...] = _dotT(s_ref[...], wd_ref[...].astype(jnp.bfloat16))

    @pl.when(e >= 1)
    def _():
        y = _dotT(h_ref[...], w2_ref[0].astype(jnp.bfloat16))
        oh_e = (jax.lax.broadcasted_iota(jnp.int32, (1, N_EXPERTS), 1)
                == e - 1)
        w_col = jnp.sum(jnp.where(oh_e, comb_ref[...], 0.0),
                        axis=-1, keepdims=True)
        out_ref[...] += w_col * y

    @pl.when(e < N_EXPERTS)
    def _():
        h = _dotT(xbf_ref[...], w1_ref[0].astype(jnp.bfloat16))
        h_ref[...] = (h * jax.nn.sigmoid(h)).astype(jnp.bfloat16)


def kernel(hidden_states, gate_w, w1, w2, shared_gate_up_w, shared_down_w):
    orig_shape = hidden_states.shape
    T = orig_shape[0] * orig_shape[1]
    x2d = hidden_states.reshape(T, HIDDEN)

    out = pl.pallas_call(
        _moe_body,
        grid=(N_EXPERTS + 1,),
        in_specs=[
            pl.BlockSpec((T, HIDDEN), lambda e: (0, 0)),
            pl.BlockSpec((N_EXPERTS, HIDDEN), lambda e: (0, 0)),
            pl.BlockSpec((1, INTER, HIDDEN),
                         lambda e: (jnp.minimum(e, N_EXPERTS - 1), 0, 0)),
            pl.BlockSpec((1, HIDDEN, INTER),
                         lambda e: (jnp.clip(e - 1, 0, N_EXPERTS - 1), 0, 0)),
            pl.BlockSpec((2 * SHARED_DIM, HIDDEN), lambda e: (0, 0)),
            pl.BlockSpec((HIDDEN, SHARED_DIM), lambda e: (0, 0)),
        ],
        out_specs=pl.BlockSpec((T, HIDDEN), lambda e: (0, 0)),
        out_shape=jax.ShapeDtypeStruct((T, HIDDEN), jnp.float32),
        scratch_shapes=[
            pltpu.VMEM((T, HIDDEN), jnp.bfloat16),
            pltpu.VMEM((T, N_EXPERTS), jnp.float32),
            pltpu.VMEM((T, INTER), jnp.bfloat16),
            pltpu.VMEM((T, SHARED_DIM), jnp.bfloat16),
        ],
    )(x2d, gate_w, w1, w2, shared_gate_up_w, shared_down_w)

    return out.reshape(orig_shape)
